# baseline (speedup 1.0000x reference)
import functools

import jax
import jax.numpy as jnp
from jax import lax
from jax.experimental import pallas as pl
from jax.experimental.pallas import tpu as pltpu
from jax.experimental.pallas import tpu_sc as plsc


CH = 64


NBUF = 3


def _sc_gather(table, idx3, n_ch, nc, row_off, NW):
    V, D = table.shape
    b_per_w = n_ch * CH
    B = NW * b_per_w
    nbuf = min(NBUF, n_ch)
    mesh = plsc.VectorSubcoreMesh(core_axis_name="c", subcore_axis_name="s")

    @functools.partial(
        pl.kernel,
        mesh=mesh,
        out_type=jax.ShapeDtypeStruct((B, D), jnp.float32),
        scratch_types=[
            pltpu.VMEM((n_ch, CH), jnp.int32),
            [pltpu.VMEM((CH, D), jnp.float32) for _ in range(nbuf)],
            [pltpu.SemaphoreType.DMA for _ in range(nbuf)],
            [pltpu.SemaphoreType.DMA for _ in range(nbuf)],
        ],
    )
    def k(table_hbm, idx_hbm, out_hbm, idx_v, bufs, gsems, osems):
        wid = lax.axis_index("s") * nc + lax.axis_index("c")
        base = wid * b_per_w
        pltpu.sync_copy(idx_hbm.at[row_off + wid], idx_v)

        def gstart(j):
            return pltpu.async_copy(
                table_hbm.at[idx_v.at[j]], bufs[j % nbuf], gsems[j % nbuf]
            )

        def ostart(j):
            return pltpu.async_copy(
                bufs[j % nbuf], out_hbm.at[pl.ds(base + j * CH, CH)], osems[j % nbuf]
            )

        gathers = [None] * n_ch
        outs = [None] * n_ch
        gathers[0] = gstart(0)
        for c in range(n_ch):
            j = c + 1
            if j < n_ch:
                if j >= nbuf:
                    outs[j - nbuf].wait()
                gathers[j] = gstart(j)
            gathers[c].wait()
            outs[c] = ostart(c)
        for c in range(max(0, n_ch - nbuf), n_ch):
            outs[c].wait()

    return k(table, idx3)


def _tc_project_part(ve_part, lid3_full, W, lang_tab, b2, blk, blk_off, B_total,
                     prev_out=None):
    Bs, D = ve_part.shape
    NL, LD = lang_tab.shape
    Gs = Bs // blk

    def body(ve_ref, lid_ref, w_ref, lt_ref, b_ref, *rest):
        out_ref = rest[-1]
        wv = w_ref[0:D, :].astype(jnp.bfloat16)
        wl = w_ref[D : D + LD, :].astype(jnp.bfloat16)
        lang_proj = (
            jnp.dot(
                lt_ref[...].astype(jnp.bfloat16), wl,
                preferred_element_type=jnp.float32,
            )
            + b_ref[...]
        )
        lid = lid_ref[0, 0, :]
        onehot = (
            lid[:, None] == lax.broadcasted_iota(jnp.int32, (blk, NL), 1)
        ).astype(jnp.bfloat16)
        out_ref[...] = jnp.dot(
            ve_ref[...].astype(jnp.bfloat16), wv,
            preferred_element_type=jnp.float32,
        ) + jnp.dot(
            onehot, lang_proj.astype(jnp.bfloat16),
            preferred_element_type=jnp.float32,
        )

    in_specs = [
        pl.BlockSpec((blk, D), lambda i: (i, 0)),
        pl.BlockSpec((1, 1, blk), lambda i, off=blk_off: (i + off, 0, 0)),
        pl.BlockSpec((D + LD, D), lambda i: (0, 0)),
        pl.BlockSpec((NL, LD), lambda i: (0, 0)),
        pl.BlockSpec((1, D), lambda i: (0, 0)),
    ]
    args = [ve_part, lid3_full, W, lang_tab, b2]
    kwargs = {}
    if prev_out is not None:
        in_specs.append(pl.BlockSpec(memory_space=pl.ANY))
        args.append(prev_out)
        kwargs["input_output_aliases"] = {5: 0}
    return pl.pallas_call(
        body,
        grid=(Gs,),
        in_specs=in_specs,
        out_specs=pl.BlockSpec((blk, D), lambda i: (i + blk_off, 0)),
        out_shape=jax.ShapeDtypeStruct((B_total, D), jnp.float32),
        **kwargs,
    )(*args)


NSPLIT = 2


def kernel(voice_id, language_id, voice_table, lang_table, W, b):
    B = voice_id.shape[0]
    V, D = voice_table.shape
    info = plsc.get_sparse_core_info()
    NW = info.num_cores * info.num_subcores
    vid = voice_id.astype(jnp.int32)
    lid = language_id.astype(jnp.int32)
    b2 = b.reshape(1, D)
    blk = 4096
    Bs = B // NSPLIT
    n_ch = Bs // (NW * CH)
    Gs = Bs // blk
    vid3 = vid.reshape(NSPLIT * NW, n_ch, CH)
    lid3 = lid.reshape(B // blk, 1, blk)
    out = None
    for s in range(NSPLIT):
        ve = _sc_gather(voice_table, vid3, n_ch, info.num_cores, s * NW, NW)
        out = _tc_project_part(
            ve, lid3, W, lang_table, b2, blk, s * Gs, B, prev_out=out
        )
    return out


# speedup vs baseline: 1.0197x; 1.0197x over previous
import functools

import jax
import jax.numpy as jnp
from jax import lax
from jax.experimental import pallas as pl
from jax.experimental.pallas import tpu as pltpu
from jax.experimental.pallas import tpu_sc as plsc


CH = 128


NBUF = 3


def _sc_gather(table, idx3, n_ch, nc, row_off, NW):
    V, D = table.shape
    b_per_w = n_ch * CH
    B = NW * b_per_w
    nbuf = min(NBUF, n_ch)
    mesh = plsc.VectorSubcoreMesh(core_axis_name="c", subcore_axis_name="s")

    @functools.partial(
        pl.kernel,
        mesh=mesh,
        out_type=jax.ShapeDtypeStruct((B, D), jnp.float32),
        scratch_types=[
            pltpu.VMEM((n_ch, CH), jnp.int32),
            [pltpu.VMEM((CH, D), jnp.float32) for _ in range(nbuf)],
            [pltpu.SemaphoreType.DMA for _ in range(nbuf)],
            [pltpu.SemaphoreType.DMA for _ in range(nbuf)],
        ],
    )
    def k(table_hbm, idx_hbm, out_hbm, idx_v, bufs, gsems, osems):
        wid = lax.axis_index("s") * nc + lax.axis_index("c")
        base = wid * b_per_w
        pltpu.sync_copy(idx_hbm.at[row_off + wid], idx_v)

        def gstart(j):
            return pltpu.async_copy(
                table_hbm.at[idx_v.at[j]], bufs[j % nbuf], gsems[j % nbuf]
            )

        def ostart(j):
            return pltpu.async_copy(
                bufs[j % nbuf], out_hbm.at[pl.ds(base + j * CH, CH)], osems[j % nbuf]
            )

        gathers = [None] * n_ch
        outs = [None] * n_ch
        gathers[0] = gstart(0)
        for c in range(n_ch):
            j = c + 1
            if j < n_ch:
                if j >= nbuf:
                    outs[j - nbuf].wait()
                gathers[j] = gstart(j)
            gathers[c].wait()
            outs[c] = ostart(c)
        for c in range(max(0, n_ch - nbuf), n_ch):
            outs[c].wait()

    return k(table, idx3)


def _tc_project_part(ve_part, lid3_full, W, lang_tab, b2, blk, blk_off, B_total,
                     prev_out=None):
    Bs, D = ve_part.shape
    NL, LD = lang_tab.shape
    Gs = Bs // blk

    def body(ve_ref, lid_ref, w_ref, lt_ref, b_ref, *rest):
        out_ref = rest[-1]
        wv = w_ref[0:D, :].astype(jnp.bfloat16)
        wl = w_ref[D : D + LD, :].astype(jnp.bfloat16)
        lang_proj = (
            jnp.dot(
                lt_ref[...].astype(jnp.bfloat16), wl,
                preferred_element_type=jnp.float32,
            )
            + b_ref[...]
        )
        lid = lid_ref[0, 0, :]
        onehot = (
            lid[:, None] == lax.broadcasted_iota(jnp.int32, (blk, NL), 1)
        ).astype(jnp.bfloat16)
        out_ref[...] = jnp.dot(
            ve_ref[...].astype(jnp.bfloat16), wv,
            preferred_element_type=jnp.float32,
        ) + jnp.dot(
            onehot, lang_proj.astype(jnp.bfloat16),
            preferred_element_type=jnp.float32,
        )

    in_specs = [
        pl.BlockSpec((blk, D), lambda i: (i, 0)),
        pl.BlockSpec((1, 1, blk), lambda i, off=blk_off: (i + off, 0, 0)),
        pl.BlockSpec((D + LD, D), lambda i: (0, 0)),
        pl.BlockSpec((NL, LD), lambda i: (0, 0)),
        pl.BlockSpec((1, D), lambda i: (0, 0)),
    ]
    args = [ve_part, lid3_full, W, lang_tab, b2]
    kwargs = {}
    if prev_out is not None:
        in_specs.append(pl.BlockSpec(memory_space=pl.ANY))
        args.append(prev_out)
        kwargs["input_output_aliases"] = {5: 0}
    return pl.pallas_call(
        body,
        grid=(Gs,),
        in_specs=in_specs,
        out_specs=pl.BlockSpec((blk, D), lambda i: (i + blk_off, 0)),
        out_shape=jax.ShapeDtypeStruct((B_total, D), jnp.float32),
        **kwargs,
    )(*args)


NSPLIT = 2


def kernel(voice_id, language_id, voice_table, lang_table, W, b):
    B = voice_id.shape[0]
    V, D = voice_table.shape
    info = plsc.get_sparse_core_info()
    NW = info.num_cores * info.num_subcores
    vid = voice_id.astype(jnp.int32)
    lid = language_id.astype(jnp.int32)
    b2 = b.reshape(1, D)
    blk = 4096
    Bs = B // NSPLIT
    n_ch = Bs // (NW * CH)
    Gs = Bs // blk
    vid3 = vid.reshape(NSPLIT * NW, n_ch, CH)
    lid3 = lid.reshape(B // blk, 1, blk)
    out = None
    for s in range(NSPLIT):
        ve = _sc_gather(voice_table, vid3, n_ch, info.num_cores, s * NW, NW)
        out = _tc_project_part(
            ve, lid3, W, lang_table, b2, blk, s * Gs, B, prev_out=out
        )
    return out
